# Initial kernel scaffold; baseline (speedup 1.0000x reference)
import jax
import jax.numpy as jnp
from jax.experimental import pallas as pl


def kernel(node_feat, edge_index, distance_matrix, nodes_to_community_tensor, params):
    raise NotImplementedError("write your pallas kernel here")


# speedup vs baseline: 13.0813x; 13.0813x over previous
import functools

import jax
import jax.numpy as jnp
from jax import lax
from jax.experimental import pallas as pl
from jax.experimental.pallas import tpu as pltpu
from jax.experimental.pallas import tpu_sc as plsc

_F32 = jnp.float32
_HI = lax.Precision.HIGHEST

_NC = 2
_NS = 16
_NW = _NC * _NS
_K = 128


def _mm(a, w):
    return lax.dot_general(a, w, (((1,), (1,)), ((), ())),
                           precision=_HI, preferred_element_type=_F32)


@functools.lru_cache(maxsize=None)
def _make_sc_deg(ep, np_rows, d):
    ew = ep // _NW
    nchunk = ew // _K
    rpt = np_rows // _NS
    mesh = plsc.VectorSubcoreMesh(core_axis_name="c", subcore_axis_name="s",
                                  num_cores=_NC, num_subcores=_NS)

    @functools.partial(
        pl.kernel, mesh=mesh,
        out_type=jax.ShapeDtypeStruct((_NC, np_rows, d), _F32),
        scratch_types=[
            pltpu.VMEM((_K,), jnp.int32),
            pltpu.VMEM((_K, d), _F32),
            pltpu.VMEM_SHARED((np_rows, d), _F32),
        ])
    def deg_kernel(dst_hbm, ones_hbm, zeros_hbm, out_hbm, didx, ones_v, acc):
        cid = lax.axis_index("c")
        sid = lax.axis_index("s")
        wid = cid * _NS + sid
        r0 = sid * rpt
        pltpu.sync_copy(zeros_hbm.at[pl.ds(r0, rpt)], acc.at[pl.ds(r0, rpt)])
        pltpu.sync_copy(ones_hbm, ones_v)
        plsc.subcore_barrier()
        base = wid * ew

        def body(j, carry):
            off = base + j * _K
            pltpu.sync_copy(dst_hbm.at[pl.ds(off, _K)], didx)
            pltpu.sync_copy(ones_v, acc.at[didx], add=True)
            return carry

        lax.fori_loop(0, nchunk, body, 0)
        plsc.subcore_barrier()
        pltpu.sync_copy(acc.at[pl.ds(r0, rpt)],
                        out_hbm.at[cid, pl.ds(r0, rpt)])

    return deg_kernel


@functools.lru_cache(maxsize=None)
def _make_sc_agg(ep, np_rows, d):
    ew = ep // _NW
    nchunk = ew // _K
    rpt = np_rows // _NS
    mesh = plsc.VectorSubcoreMesh(core_axis_name="c", subcore_axis_name="s",
                                  num_cores=_NC, num_subcores=_NS)

    @functools.partial(
        pl.kernel, mesh=mesh,
        out_type=jax.ShapeDtypeStruct((_NC, np_rows, d), _F32),
        scratch_types=[
            pltpu.VMEM((_K,), jnp.int32),
            pltpu.VMEM((_K,), jnp.int32),
            pltpu.VMEM((_K, d), _F32),
            pltpu.VMEM_SHARED((np_rows, d), _F32),
            pltpu.SemaphoreType.DMA,
        ])
    def agg_kernel(x_hbm, src_hbm, dst_hbm, zeros_hbm, out_hbm,
                   sidx, didx, rows, acc, sem):
        cid = lax.axis_index("c")
        sid = lax.axis_index("s")
        wid = cid * _NS + sid
        r0 = sid * rpt
        pltpu.sync_copy(zeros_hbm.at[pl.ds(r0, rpt)], acc.at[pl.ds(r0, rpt)])
        plsc.subcore_barrier()
        base = wid * ew

        def body(j, carry):
            off = base + j * _K
            pltpu.sync_copy(src_hbm.at[pl.ds(off, _K)], sidx)
            pltpu.sync_copy(dst_hbm.at[pl.ds(off, _K)], didx)
            pltpu.async_copy(x_hbm.at[sidx], rows, sem).wait()
            pltpu.sync_copy(rows, acc.at[didx], add=True)
            return carry

        lax.fori_loop(0, nchunk, body, 0)
        plsc.subcore_barrier()
        pltpu.sync_copy(acc.at[pl.ds(r0, rpt)],
                        out_hbm.at[cid, pl.ds(r0, rpt)])

    return agg_kernel


def _sc_deg(dstp, ones_kd, zeros_nd, ep, np_rows):
    return _make_sc_deg(ep, np_rows, ones_kd.shape[1])(dstp, ones_kd, zeros_nd)


def _sc_agg(x, srcp, dstp, zeros_nd, ep, np_rows):
    return _make_sc_agg(ep, np_rows, x.shape[1])(x, srcp, dstp, zeros_nd)


def _fc_in_body(nf_ref, w1_ref, b1_ref, w2_ref, b2_ref, o_ref):
    x = jnp.maximum(_mm(nf_ref[...], w1_ref[...]) + b1_ref[...], 0.0)
    o_ref[...] = _mm(x, w2_ref[...]) + b2_ref[...]


def _segstats_body(x_ref, comm_ref, sums_ref, cnt_ref):
    i = pl.program_id(0)

    @pl.when(i == 0)
    def _init():
        sums_ref[...] = jnp.zeros_like(sums_ref)
        cnt_ref[...] = jnp.zeros_like(cnt_ref)

    comm = comm_ref[0, 0, :]
    c = cnt_ref.shape[1]
    onehot = (comm[:, None] == lax.broadcasted_iota(jnp.int32, (1, c), 1)
              ).astype(_F32)
    sums_ref[...] += lax.dot_general(onehot, x_ref[...],
                                     (((0,), (0,)), ((), ())),
                                     precision=_HI,
                                     preferred_element_type=_F32)
    cnt_ref[...] += jnp.sum(onehot, axis=0, keepdims=True)


def _attn_body(x_ref, dmi_ref, sums_ref, cnt_ref, emb_ref, wdis_ref, bdis_ref,
               wp_ref, bp_ref, wq_ref, bq_ref, wk_ref, bk_ref, wv_ref, bv_ref,
               wf1_ref, bf1_ref, wf2_ref, bf2_ref, o_ref, *, heads, n_dis):
    cnt = cnt_ref[0, :]
    sizes = jnp.maximum(cnt, 1.0)
    avg = sums_ref[...] / sizes[:, None]
    score = jnp.sum(emb_ref[...] * wdis_ref[...], axis=1)
    dmi = dmi_ref[...]
    dm = jnp.zeros(dmi.shape, _F32)
    for v in range(n_dis):
        dm = dm + jnp.where(dmi == v, score[v], 0.0)
    bias = dm + bdis_ref[...] + jnp.log(cnt)[None, :]

    x = x_ref[...]
    qx = _mm(x, wp_ref[...]) + bp_ref[...]
    qq = _mm(qx, wq_ref[...]) + bq_ref[...]
    kk = _mm(avg, wk_ref[...]) + bk_ref[...]
    vv = _mm(avg, wv_ref[...]) + bv_ref[...]

    hd = qq.shape[1] // heads
    scale = 1.0 / (hd ** 0.5)
    outs = []
    for hh in range(heads):
        sl = slice(hh * hd, (hh + 1) * hd)
        qh, kh, vh = qq[:, sl], kk[:, sl], vv[:, sl]
        dots = lax.dot_general(qh, kh, (((1,), (1,)), ((), ())),
                               precision=_HI,
                               preferred_element_type=_F32) * scale + bias
        m = jnp.max(dots, axis=1, keepdims=True)
        e = jnp.exp(dots - m)
        s = jnp.sum(e, axis=1, keepdims=True)
        outs.append(lax.dot_general(e / s, vh, (((1,), (0,)), ((), ())),
                                    precision=_HI,
                                    preferred_element_type=_F32))
    att = jnp.concatenate(outs, axis=1)
    h1 = jnp.maximum(_mm(att, wf1_ref[...]) + bf1_ref[...], 0.0)
    o_ref[...] = jnp.maximum(_mm(h1, wf2_ref[...]) + bf2_ref[...], 0.0)


def _dinv_of(degp):
    deg = degp[0, :, 0] + degp[1, :, 0] + 1.0
    return lax.rsqrt(deg)


def _prescale_body(degp_ref, nf_ref, o_ref):
    dinv = _dinv_of(degp_ref[...])
    o_ref[...] = nf_ref[...] * dinv[:, None]


def _gcn_mid_body(agg_ref, degp_ref, xs_ref, wg_ref, bg_ref, o_ref):
    dinv = _dinv_of(degp_ref[...])
    a = agg_ref[0] + agg_ref[1] + xs_ref[...]
    g = a * dinv[:, None]
    hg = jnp.maximum(_mm(g, wg_ref[...]) + bg_ref[...], 0.0)
    o_ref[...] = hg * dinv[:, None]


def _final_body(xt_ref, agg_ref, degp_ref, xs_ref, wg_ref, bg_ref,
                wo_ref, bo_ref, o_ref):
    dinv = _dinv_of(degp_ref[...])
    a = agg_ref[0] + agg_ref[1] + xs_ref[...]
    g = a * dinv[:, None]
    xg = _mm(g, wg_ref[...]) + bg_ref[...]
    o_ref[...] = _mm(xt_ref[...] + xg, wo_ref[...]) + bo_ref[...]


def _row_spec(b, d):
    return pl.BlockSpec((b, d), lambda i: (i, 0))


def _full_spec(shape):
    nz = (0,) * len(shape)
    return pl.BlockSpec(shape, lambda i, _nz=nz: _nz)


def kernel(node_feat, edge_index, distance_matrix, nodes_to_community_tensor,
           params):
    n, d_in = node_feat.shape
    e = edge_index.shape[1]
    h = params["fc_in"]["W1"].shape[0]
    c = distance_matrix.shape[1]
    heads = 4
    n_dis = params["layers"][0]["emb_dis"].shape[0]

    blk = 1000 if n % 1000 == 0 else 8
    nblk = n // blk

    ew = -(-e // (_NW * _K)) * _K
    ep = ew * _NW
    np_rows = -(-(n + 8) // (_NS * 8)) * (_NS * 8)
    pad = ep - e
    src = edge_index[0].astype(jnp.int32)
    dst = edge_index[1].astype(jnp.int32)
    srcp = jnp.concatenate([src, jnp.zeros((pad,), jnp.int32)])
    dstp = jnp.concatenate([dst, jnp.full((pad,), n, jnp.int32)])
    zeros_nd = jnp.zeros((np_rows, h), _F32)
    ones_kd = jnp.ones((_K, h), _F32)
    comm = nodes_to_community_tensor.astype(jnp.int32)
    comm3 = comm.reshape(nblk, 1, blk)
    dmi = distance_matrix.astype(jnp.int32)

    def b2(v):
        return v.reshape(1, -1)

    degp = _sc_deg(dstp, ones_kd, zeros_nd, ep, np_rows)

    deg_spec = pl.BlockSpec((_NC, blk, h), lambda i: (0, i, 0))
    agg_spec = pl.BlockSpec((_NC, blk, h), lambda i: (0, i, 0))

    xs1 = pl.pallas_call(
        _prescale_body, grid=(nblk,),
        in_specs=[deg_spec, _row_spec(blk, d_in)],
        out_specs=_row_spec(blk, d_in),
        out_shape=jax.ShapeDtypeStruct((n, d_in), _F32),
    )(degp, node_feat)

    agg1 = _sc_agg(xs1, srcp, dstp, zeros_nd, ep, np_rows)

    g = params["gnn"]
    xs2 = pl.pallas_call(
        _gcn_mid_body, grid=(nblk,),
        in_specs=[agg_spec, deg_spec, _row_spec(blk, h),
                  _full_spec((h, d_in)), _full_spec((1, h))],
        out_specs=_row_spec(blk, h),
        out_shape=jax.ShapeDtypeStruct((n, h), _F32),
    )(agg1, degp, xs1, g["W1"], b2(g["b1"]))

    agg2 = _sc_agg(xs2, srcp, dstp, zeros_nd, ep, np_rows)

    fi = params["fc_in"]
    x = pl.pallas_call(
        _fc_in_body, grid=(nblk,),
        in_specs=[_row_spec(blk, d_in), _full_spec((h, d_in)),
                  _full_spec((1, h)), _full_spec((h, h)), _full_spec((1, h))],
        out_specs=_row_spec(blk, h),
        out_shape=jax.ShapeDtypeStruct((n, h), _F32),
    )(node_feat, fi["W1"], b2(fi["b1"]), fi["W2"], b2(fi["b2"]))

    for p in params["layers"]:
        sums, cnt = pl.pallas_call(
            _segstats_body, grid=(nblk,),
            in_specs=[_row_spec(blk, h),
                      pl.BlockSpec((1, 1, blk), lambda i: (i, 0, 0))],
            out_specs=[_full_spec((c, h)), _full_spec((1, c))],
            out_shape=[jax.ShapeDtypeStruct((c, h), _F32),
                       jax.ShapeDtypeStruct((1, c), _F32)],
        )(x, comm3)

        emb_pad = jnp.concatenate(
            [p["emb_dis"], jnp.zeros((32 - n_dis, h), _F32)], axis=0)
        body = functools.partial(_attn_body, heads=heads, n_dis=n_dis)
        x = pl.pallas_call(
            body, grid=(nblk,),
            in_specs=[_row_spec(blk, h), _row_spec(blk, c),
                      _full_spec((c, h)), _full_spec((1, c)),
                      _full_spec((32, h)), _full_spec((1, h)),
                      _full_spec((1, 1)),
                      _full_spec((h, h)), _full_spec((1, h)),
                      _full_spec((h, h)), _full_spec((1, h)),
                      _full_spec((h, h)), _full_spec((1, h)),
                      _full_spec((h, h)), _full_spec((1, h)),
                      _full_spec((h, h)), _full_spec((1, h)),
                      _full_spec((h, h)), _full_spec((1, h))],
            out_specs=_row_spec(blk, h),
            out_shape=jax.ShapeDtypeStruct((n, h), _F32),
        )(x, dmi, sums, cnt, emb_pad, p["w_dis"],
          p["b_dis"].reshape(1, 1),
          p["Wp"], b2(p["bp"]), p["Wq"], b2(p["bq"]),
          p["Wk"], b2(p["bk"]), p["Wv"], b2(p["bv"]),
          p["Wf1"], b2(p["bf1"]), p["Wf2"], b2(p["bf2"]))

    fo = params["fc_out"]
    out = pl.pallas_call(
        _final_body, grid=(nblk,),
        in_specs=[_row_spec(blk, h), agg_spec, deg_spec, _row_spec(blk, h),
                  _full_spec((h, h)), _full_spec((1, h)),
                  _full_spec((fo["W"].shape[0], h)),
                  _full_spec((1, fo["W"].shape[0]))],
        out_specs=_row_spec(blk, fo["W"].shape[0]),
        out_shape=jax.ShapeDtypeStruct((n, fo["W"].shape[0]), _F32),
    )(x, agg2, degp, xs2, g["W2"], b2(g["b2"]), fo["W"], b2(fo["b"]))

    return out
